# baseline (speedup 1.0000x reference)
import jax
import jax.numpy as jnp
from jax import lax
from jax.experimental import pallas as pl
from jax.experimental.pallas import tpu as pltpu

_K = 1000
_KP = 1024
_CONF = 0.1
_IOU = 0.7
_SCALE = min(640.0 / 1080.0, 640.0 / 1920.0)
_ROWB = 128
_CB = 48
_CH = 1024


def _score_body(p_ref, m_ref, b_ref, conf_ref, s0_ref):
    c = pl.program_id(1)
    p = p_ref[0]
    rg = c * _CB + lax.broadcasted_iota(jnp.int32, p.shape, 0)
    part = jnp.max(jnp.where((rg >= 4) & (rg < 84), p, -1e30),
                   axis=0, keepdims=True)

    @pl.when(c == 0)
    def _():
        b_ref[0] = p[0:4]
        s0_ref[:, :] = p[4:5, :]
        conf_ref[:, :] = part

    @pl.when(c > 0)
    def _():
        conf_ref[:, :] = jnp.maximum(conf_ref[:, :], part)

    @pl.when(c == pl.num_programs(1) - 1)
    def _():
        s0 = s0_ref[:, :]
        valid = (s0 >= conf_ref[:, :]) & (s0 > _CONF)
        m_ref[0] = jnp.where(valid, s0, -1.0)


def _nms_body(bx_ref, idx_ref, s_ref, out_ref, box_ref, m_ref, kk_ref):
    bx = bx_ref[0]
    idxr = idx_ref[0]
    s = s_ref[0]
    a_dim = bx.shape[1]

    vb = s > _CONF
    keep0 = vb.astype(jnp.float32)
    nv = jnp.sum(keep0).astype(jnp.int32)

    for kb in range(_KP // _ROWB):
        k0 = kb * _ROWB

        @pl.when(nv > k0)
        def _():
            idxs = idxr[:, k0:k0 + _ROWB]
            bacc = jnp.zeros((4, _ROWB), jnp.float32)
            for base in range(0, a_dim, _CH):
                w = min(_CH, a_dim - base)
                ioc = base + lax.broadcasted_iota(jnp.int32, (w, _ROWB), 0)
                oh = (ioc == idxs).astype(jnp.float32)
                bacc = bacc + lax.dot_general(
                    bx[:, base:base + w], oh, (((1,), (0,)), ((), ())),
                    precision=lax.Precision.HIGHEST,
                    preferred_element_type=jnp.float32)
            box_ref[:, k0:k0 + _ROWB] = jnp.concatenate(
                [bacc, jnp.zeros((4, _ROWB), jnp.float32)], axis=0)

    bvals = box_ref[:, :]
    cx = jnp.where(vb, bvals[0:1], 0.0)
    cy = jnp.where(vb, bvals[1:2], 0.0)
    ww = jnp.where(vb, bvals[2:3], 0.0)
    hh = jnp.where(vb, bvals[3:4], 0.0)
    x1r = cx - ww * 0.5
    y1r = cy - hh * 0.5
    x2r = cx + ww * 0.5
    y2r = cy + hh * 0.5
    ar_r = (x2r - x1r) * (y2r - y1r)

    x1c = jnp.transpose(x1r, (1, 0))
    y1c = jnp.transpose(y1r, (1, 0))
    x2c = jnp.transpose(x2r, (1, 0))
    y2c = jnp.transpose(y2r, (1, 0))
    ar_c = jnp.transpose(ar_r, (1, 0))

    for rb in range(_KP // _ROWB):
        r0 = rb * _ROWB

        @pl.when(nv > r0)
        def _():
            xl = jnp.maximum(x1c[r0:r0 + _ROWB], x1r)
            yt = jnp.maximum(y1c[r0:r0 + _ROWB], y1r)
            xr = jnp.minimum(x2c[r0:r0 + _ROWB], x2r)
            yb = jnp.minimum(y2c[r0:r0 + _ROWB], y2r)
            iw = jnp.maximum(xr - xl, 0.0)
            ih = jnp.maximum(yb - yt, 0.0)
            inter = iw * ih
            iou = inter / (ar_c[r0:r0 + _ROWB] + ar_r - inter + 1e-7)
            gi = r0 + lax.broadcasted_iota(jnp.int32, (_ROWB, _KP), 0)
            gj = lax.broadcasted_iota(jnp.int32, (_ROWB, _KP), 1)
            m_ref[r0:r0 + _ROWB, :] = ((iou > _IOU) & (gi < gj)).astype(
                jnp.float32)

    def step(kv):
        kk_ref[:, :] = jnp.zeros((1, _KP), jnp.float32)
        for rc in range(_KP // _ROWB):
            r0 = rc * _ROWB

            @pl.when(nv > r0)
            def _():
                kk_ref[:, :] += lax.dot_general(
                    kv[:, r0:r0 + _ROWB], m_ref[r0:r0 + _ROWB, :],
                    (((1,), (0,)), ((), ())),
                    preferred_element_type=jnp.float32)
        return jnp.where(kk_ref[:, :] > 0.0, 0.0, keep0)

    def cond_f(c):
        prev, cur = c
        return jnp.any(prev != cur)

    def body_f(c):
        prev, cur = c
        return (cur, step(cur))

    keep = lax.while_loop(cond_f, body_f, (keep0, step(keep0)))[1]

    zero = jnp.zeros_like(s)
    outt = jnp.concatenate(
        [x1r / _SCALE, y1r / _SCALE, x2r / _SCALE, y2r / _SCALE,
         s, zero, zero, zero], axis=0) * keep
    out_ref[0] = outt


def kernel(preds):
    b_dim, c_dim, a_dim = preds.shape

    masked, boxes4 = pl.pallas_call(
        _score_body,
        grid=(b_dim, pl.cdiv(c_dim, _CB)),
        in_specs=[pl.BlockSpec((1, _CB, a_dim), lambda b, c: (b, c, 0))],
        out_specs=[
            pl.BlockSpec((1, 1, a_dim), lambda b, c: (b, 0, 0)),
            pl.BlockSpec((1, 4, a_dim), lambda b, c: (b, 0, 0)),
        ],
        out_shape=[
            jax.ShapeDtypeStruct((b_dim, 1, a_dim), jnp.float32),
            jax.ShapeDtypeStruct((b_dim, 4, a_dim), jnp.float32),
        ],
        scratch_shapes=[
            pltpu.VMEM((1, a_dim), jnp.float32),
            pltpu.VMEM((1, a_dim), jnp.float32),
        ],
        compiler_params=pltpu.CompilerParams(
            dimension_semantics=("parallel", "arbitrary")),
    )(preds)

    masked2 = masked.reshape(b_dim, a_dim)
    grp = 100
    n_grp = a_dim // grp
    mg = masked2.reshape(b_dim, n_grp, grp)
    cnt_max = jnp.max(jnp.sum((mg > _CONF).astype(jnp.int32), axis=2))

    def _topk_fast(_):
        gs, gi = lax.top_k(mg, 16)
        gidx = gi + jnp.arange(n_grp, dtype=gi.dtype)[None, :, None] * grp
        nk, ni = lax.sort(
            [-gs.reshape(b_dim, n_grp * 16), gidx.reshape(b_dim, n_grp * 16)],
            num_keys=1, is_stable=True)
        return -nk[:, :_K], ni[:, :_K]

    def _topk_full(_):
        fs, fi = lax.top_k(masked2, _K)
        return fs, fi

    top_s, idx = lax.cond(cnt_max <= 16, _topk_fast, _topk_full, None)
    idx_p = jnp.pad(idx, ((0, 0), (0, _KP - _K)),
                    constant_values=-1).reshape(b_dim, 1, _KP)
    s_p = jnp.pad(top_s, ((0, 0), (0, _KP - _K)),
                  constant_values=-1.0).reshape(b_dim, 1, _KP)

    out = pl.pallas_call(
        _nms_body,
        grid=(b_dim,),
        in_specs=[
            pl.BlockSpec((1, 4, a_dim), lambda b: (b, 0, 0)),
            pl.BlockSpec((1, 1, _KP), lambda b: (b, 0, 0)),
            pl.BlockSpec((1, 1, _KP), lambda b: (b, 0, 0)),
        ],
        out_specs=pl.BlockSpec((1, 8, _KP), lambda b: (b, 0, 0)),
        out_shape=jax.ShapeDtypeStruct((b_dim, 8, _KP), jnp.float32),
        scratch_shapes=[
            pltpu.VMEM((8, _KP), jnp.float32),
            pltpu.VMEM((_KP, _KP), jnp.float32),
            pltpu.VMEM((1, _KP), jnp.float32),
        ],
        compiler_params=pltpu.CompilerParams(
            dimension_semantics=("parallel",)),
    )(boxes4, idx_p, s_p)

    return jnp.transpose(out, (0, 2, 1))[:, :_K, :6]


# speedup vs baseline: 1.0568x; 1.0568x over previous
import jax
import jax.numpy as jnp
from jax import lax
from jax.experimental import pallas as pl
from jax.experimental.pallas import tpu as pltpu

_K = 1000
_KP = 1024
_CONF = 0.1
_IOU = 0.7
_SCALE = min(640.0 / 1080.0, 640.0 / 1920.0)
_ROWB = 128
_CH = 1024


def _score_body(p_ref, m_ref, b_ref):
    p = p_ref[0]
    conf = jnp.maximum(
        jnp.max(p[8:, :], axis=0, keepdims=True),
        jnp.max(p[4:8, :], axis=0, keepdims=True))
    s0 = p[4:5, :]
    valid = (s0 >= conf) & (s0 > _CONF)
    m_ref[0] = jnp.where(valid, s0, -1.0)
    b_ref[0] = p[0:4]


def _nms_body(bx_ref, idx_ref, s_ref, out_ref, box_ref, m_ref, kk_ref):
    bx = bx_ref[0]
    idxr = idx_ref[0]
    s = s_ref[0]
    a_dim = bx.shape[1]

    vb = s > _CONF
    keep0 = vb.astype(jnp.float32)
    nv = jnp.sum(keep0).astype(jnp.int32)

    for kb in range(_KP // _ROWB):
        k0 = kb * _ROWB

        @pl.when(nv > k0)
        def _():
            idxs = idxr[:, k0:k0 + _ROWB]
            bacc = jnp.zeros((4, _ROWB), jnp.float32)
            for base in range(0, a_dim, _CH):
                w = min(_CH, a_dim - base)
                ioc = base + lax.broadcasted_iota(jnp.int32, (w, _ROWB), 0)
                oh = (ioc == idxs).astype(jnp.float32)
                bacc = bacc + lax.dot_general(
                    bx[:, base:base + w], oh, (((1,), (0,)), ((), ())),
                    precision=lax.Precision.HIGHEST,
                    preferred_element_type=jnp.float32)
            box_ref[:, k0:k0 + _ROWB] = jnp.concatenate(
                [bacc, jnp.zeros((4, _ROWB), jnp.float32)], axis=0)

    bvals = box_ref[:, :]
    cx = jnp.where(vb, bvals[0:1], 0.0)
    cy = jnp.where(vb, bvals[1:2], 0.0)
    ww = jnp.where(vb, bvals[2:3], 0.0)
    hh = jnp.where(vb, bvals[3:4], 0.0)
    x1r = cx - ww * 0.5
    y1r = cy - hh * 0.5
    x2r = cx + ww * 0.5
    y2r = cy + hh * 0.5
    ar_r = (x2r - x1r) * (y2r - y1r)

    x1c = jnp.transpose(x1r, (1, 0))
    y1c = jnp.transpose(y1r, (1, 0))
    x2c = jnp.transpose(x2r, (1, 0))
    y2c = jnp.transpose(y2r, (1, 0))
    ar_c = jnp.transpose(ar_r, (1, 0))

    for rb in range(_KP // _ROWB):
        r0 = rb * _ROWB

        @pl.when(nv > r0)
        def _():
            xl = jnp.maximum(x1c[r0:r0 + _ROWB], x1r)
            yt = jnp.maximum(y1c[r0:r0 + _ROWB], y1r)
            xr = jnp.minimum(x2c[r0:r0 + _ROWB], x2r)
            yb = jnp.minimum(y2c[r0:r0 + _ROWB], y2r)
            iw = jnp.maximum(xr - xl, 0.0)
            ih = jnp.maximum(yb - yt, 0.0)
            inter = iw * ih
            iou = inter / (ar_c[r0:r0 + _ROWB] + ar_r - inter + 1e-7)
            gi = r0 + lax.broadcasted_iota(jnp.int32, (_ROWB, _KP), 0)
            gj = lax.broadcasted_iota(jnp.int32, (_ROWB, _KP), 1)
            m_ref[r0:r0 + _ROWB, :] = ((iou > _IOU) & (gi < gj)).astype(
                jnp.float32)

    def step(kv):
        kk_ref[:, :] = jnp.zeros((1, _KP), jnp.float32)
        for rc in range(_KP // _ROWB):
            r0 = rc * _ROWB

            @pl.when(nv > r0)
            def _():
                kk_ref[:, :] += lax.dot_general(
                    kv[:, r0:r0 + _ROWB], m_ref[r0:r0 + _ROWB, :],
                    (((1,), (0,)), ((), ())),
                    preferred_element_type=jnp.float32)
        return jnp.where(kk_ref[:, :] > 0.0, 0.0, keep0)

    def cond_f(c):
        prev, cur = c
        return jnp.any(prev != cur)

    def body_f(c):
        prev, cur = c
        return (cur, step(cur))

    keep = lax.while_loop(cond_f, body_f, (keep0, step(keep0)))[1]

    zero = jnp.zeros_like(s)
    outt = jnp.concatenate(
        [x1r / _SCALE, y1r / _SCALE, x2r / _SCALE, y2r / _SCALE,
         s, zero, zero, zero], axis=0) * keep
    out_ref[0] = outt


def kernel(preds):
    b_dim, c_dim, a_dim = preds.shape

    masked, boxes4 = pl.pallas_call(
        _score_body,
        grid=(b_dim,),
        in_specs=[pl.BlockSpec((1, c_dim, a_dim), lambda b: (b, 0, 0))],
        out_specs=[
            pl.BlockSpec((1, 1, a_dim), lambda b: (b, 0, 0)),
            pl.BlockSpec((1, 4, a_dim), lambda b: (b, 0, 0)),
        ],
        out_shape=[
            jax.ShapeDtypeStruct((b_dim, 1, a_dim), jnp.float32),
            jax.ShapeDtypeStruct((b_dim, 4, a_dim), jnp.float32),
        ],
        compiler_params=pltpu.CompilerParams(
            dimension_semantics=("parallel",)),
    )(preds)

    masked2 = masked.reshape(b_dim, a_dim)
    grp = 100
    n_grp = a_dim // grp
    mg = masked2.reshape(b_dim, n_grp, grp)
    cnt_max = jnp.max(jnp.sum((mg > _CONF).astype(jnp.int32), axis=2))

    def _topk_fast(_):
        gs, gi = lax.top_k(mg, 16)
        gidx = gi + jnp.arange(n_grp, dtype=gi.dtype)[None, :, None] * grp
        nk, ni = lax.sort(
            [-gs.reshape(b_dim, n_grp * 16), gidx.reshape(b_dim, n_grp * 16)],
            num_keys=1, is_stable=True)
        return -nk[:, :_K], ni[:, :_K]

    def _topk_full(_):
        fs, fi = lax.top_k(masked2, _K)
        return fs, fi

    top_s, idx = lax.cond(cnt_max <= 16, _topk_fast, _topk_full, None)
    idx_p = jnp.pad(idx, ((0, 0), (0, _KP - _K)),
                    constant_values=-1).reshape(b_dim, 1, _KP)
    s_p = jnp.pad(top_s, ((0, 0), (0, _KP - _K)),
                  constant_values=-1.0).reshape(b_dim, 1, _KP)

    out = pl.pallas_call(
        _nms_body,
        grid=(b_dim,),
        in_specs=[
            pl.BlockSpec((1, 4, a_dim), lambda b: (b, 0, 0)),
            pl.BlockSpec((1, 1, _KP), lambda b: (b, 0, 0)),
            pl.BlockSpec((1, 1, _KP), lambda b: (b, 0, 0)),
        ],
        out_specs=pl.BlockSpec((1, 8, _KP), lambda b: (b, 0, 0)),
        out_shape=jax.ShapeDtypeStruct((b_dim, 8, _KP), jnp.float32),
        scratch_shapes=[
            pltpu.VMEM((8, _KP), jnp.float32),
            pltpu.VMEM((_KP, _KP), jnp.float32),
            pltpu.VMEM((1, _KP), jnp.float32),
        ],
        compiler_params=pltpu.CompilerParams(
            dimension_semantics=("parallel",)),
    )(boxes4, idx_p, s_p)

    return jnp.transpose(out, (0, 2, 1))[:, :_K, :6]
